# baseline (speedup 1.0000x reference)
import jax
import jax.numpy as jnp
from jax.experimental import pallas as pl
from jax.experimental.pallas import tpu as pltpu

BM = 256


def _p1_kernel(x_ref, w1_ref, s1_ref):
    s1_ref[...] = jnp.dot(x_ref[...], w1_ref[...],
                          preferred_element_type=jnp.float32)


def _p2_kernel(adj_ref, s1_ref, b1_ref, d1_ref, w2_ref, s2_ref):
    h = jnp.dot(adj_ref[...], s1_ref[...], preferred_element_type=jnp.float32)
    h = jnp.maximum(h + b1_ref[...], 0.0) * d1_ref[...]
    s2_ref[...] = jnp.dot(h, w2_ref[...], preferred_element_type=jnp.float32)


def _p3_kernel(adj_ref, s2_ref, b2_ref, d2_ref, w3_ref, s3_ref):
    h = jnp.dot(adj_ref[...], s2_ref[...], preferred_element_type=jnp.float32)
    h = (h + b2_ref[...]) * d2_ref[...]
    s3_ref[...] = jnp.dot(h, w3_ref[...], preferred_element_type=jnp.float32)


def _p4_kernel(adj_ref, s3_ref, b3_ref, o_ref):
    h = jnp.dot(adj_ref[...], s3_ref[...], preferred_element_type=jnp.float32)
    h = h + b3_ref[...]
    m = jnp.max(h, axis=1, keepdims=True)
    s = jnp.log(jnp.sum(jnp.exp(h - m), axis=1, keepdims=True))
    o_ref[...] = (h - m) - s


def _row_block(d1):
    return pl.BlockSpec((BM, d1), lambda i: (i, 0))


def _whole(shape):
    return pl.BlockSpec(shape, lambda i: (0,) * len(shape))


def kernel(x, adj, W1, b1, W2, b2, W3, b3):
    n, d_in = x.shape
    d_h1 = W1.shape[1]
    d_h2 = W2.shape[1]
    d_out = W3.shape[1]
    grid = (pl.cdiv(n, BM),)

    m1 = jax.random.bernoulli(jax.random.key(101), 0.5, (n, d_h1))
    m2 = jax.random.bernoulli(jax.random.key(202), 0.5, (n, d_h2))
    d1 = jnp.where(m1, jnp.float32(2.0), jnp.float32(0.0))
    d2 = jnp.where(m2, jnp.float32(2.0), jnp.float32(0.0))

    b1r = b1.reshape(1, d_h1)
    b2r = b2.reshape(1, d_h2)
    b3r = b3.reshape(1, d_out)

    s1 = pl.pallas_call(
        _p1_kernel,
        grid=grid,
        in_specs=[_row_block(d_in), _whole(W1.shape)],
        out_specs=_row_block(d_h1),
        out_shape=jax.ShapeDtypeStruct((n, d_h1), jnp.float32),
    )(x, W1)

    s2 = pl.pallas_call(
        _p2_kernel,
        grid=grid,
        in_specs=[_row_block(n), _whole(s1.shape), _whole(b1r.shape),
                  _row_block(d_h1), _whole(W2.shape)],
        out_specs=_row_block(d_h2),
        out_shape=jax.ShapeDtypeStruct((n, d_h2), jnp.float32),
    )(adj, s1, b1r, d1, W2)

    s3 = pl.pallas_call(
        _p3_kernel,
        grid=grid,
        in_specs=[_row_block(n), _whole(s2.shape), _whole(b2r.shape),
                  _row_block(d_h2), _whole(W3.shape)],
        out_specs=_row_block(d_out),
        out_shape=jax.ShapeDtypeStruct((n, d_out), jnp.float32),
    )(adj, s2, b2r, d2, W3)

    out = pl.pallas_call(
        _p4_kernel,
        grid=grid,
        in_specs=[_row_block(n), _whole(s3.shape), _whole(b3r.shape)],
        out_specs=_row_block(d_out),
        out_shape=jax.ShapeDtypeStruct((n, d_out), jnp.float32),
    )(adj, s3, b3r)

    return out


# speedup vs baseline: 1.0132x; 1.0132x over previous
import jax
import jax.numpy as jnp
from jax.experimental import pallas as pl
from jax.experimental.pallas import tpu as pltpu

BM = 512


def _p1_kernel(x_ref, w1_ref, s1_ref):
    s1_ref[...] = jnp.dot(x_ref[...], w1_ref[...],
                          preferred_element_type=jnp.float32)


def _p2_kernel(adj_ref, s1_ref, b1_ref, d1_ref, w2_ref, s2_ref):
    h = jnp.dot(adj_ref[...], s1_ref[...], preferred_element_type=jnp.float32)
    h = jnp.maximum(h + b1_ref[...], 0.0) * d1_ref[...]
    s2_ref[...] = jnp.dot(h, w2_ref[...], preferred_element_type=jnp.float32)


def _p3_kernel(adj_ref, s2_ref, b2_ref, d2_ref, w3_ref, s3_ref):
    h = jnp.dot(adj_ref[...], s2_ref[...], preferred_element_type=jnp.float32)
    h = (h + b2_ref[...]) * d2_ref[...]
    s3_ref[...] = jnp.dot(h, w3_ref[...], preferred_element_type=jnp.float32)


def _p4_kernel(adj_ref, s3_ref, b3_ref, o_ref):
    h = jnp.dot(adj_ref[...], s3_ref[...], preferred_element_type=jnp.float32)
    h = h + b3_ref[...]
    m = jnp.max(h, axis=1, keepdims=True)
    s = jnp.log(jnp.sum(jnp.exp(h - m), axis=1, keepdims=True))
    o_ref[...] = (h - m) - s


def _row_block(d1):
    return pl.BlockSpec((BM, d1), lambda i: (i, 0))


def _whole(shape):
    return pl.BlockSpec(shape, lambda i: (0,) * len(shape))


def kernel(x, adj, W1, b1, W2, b2, W3, b3):
    n, d_in = x.shape
    d_h1 = W1.shape[1]
    d_h2 = W2.shape[1]
    d_out = W3.shape[1]
    grid = (pl.cdiv(n, BM),)

    m1 = jax.random.bernoulli(jax.random.key(101), 0.5, (n, d_h1))
    m2 = jax.random.bernoulli(jax.random.key(202), 0.5, (n, d_h2))
    d1 = jnp.where(m1, jnp.float32(2.0), jnp.float32(0.0))
    d2 = jnp.where(m2, jnp.float32(2.0), jnp.float32(0.0))

    b1r = b1.reshape(1, d_h1)
    b2r = b2.reshape(1, d_h2)
    b3r = b3.reshape(1, d_out)

    s1 = pl.pallas_call(
        _p1_kernel,
        grid=grid,
        in_specs=[_row_block(d_in), _whole(W1.shape)],
        out_specs=_row_block(d_h1),
        out_shape=jax.ShapeDtypeStruct((n, d_h1), jnp.float32),
    )(x, W1)

    s2 = pl.pallas_call(
        _p2_kernel,
        grid=grid,
        in_specs=[_row_block(n), _whole(s1.shape), _whole(b1r.shape),
                  _row_block(d_h1), _whole(W2.shape)],
        out_specs=_row_block(d_h2),
        out_shape=jax.ShapeDtypeStruct((n, d_h2), jnp.float32),
    )(adj, s1, b1r, d1, W2)

    s3 = pl.pallas_call(
        _p3_kernel,
        grid=grid,
        in_specs=[_row_block(n), _whole(s2.shape), _whole(b2r.shape),
                  _row_block(d_h2), _whole(W3.shape)],
        out_specs=_row_block(d_out),
        out_shape=jax.ShapeDtypeStruct((n, d_out), jnp.float32),
    )(adj, s2, b2r, d2, W3)

    out = pl.pallas_call(
        _p4_kernel,
        grid=grid,
        in_specs=[_row_block(n), _whole(s3.shape), _whole(b3r.shape)],
        out_specs=_row_block(d_out),
        out_shape=jax.ShapeDtypeStruct((n, d_out), jnp.float32),
    )(adj, s3, b3r)

    return out
